# Initial kernel scaffold; baseline (speedup 1.0000x reference)
import jax
import jax.numpy as jnp
from jax.experimental import pallas as pl


def kernel(x, W1, b1, W2, b2):
    raise NotImplementedError("write your pallas kernel here")


# speedup vs baseline: 1.1746x; 1.1746x over previous
import functools

import jax
import jax.numpy as jnp
from jax import lax
from jax.experimental import pallas as pl
from jax.experimental.pallas import tpu as pltpu
from jax.experimental.pallas import tpu_sc as plsc

HIDDEN = 128
TARGET_RATIO = 0.5
MIN_SEQ_LEN = 2048
LANES = 16
GCH = 64

_I32_MIN = -2147483648


def _params_body(k_keep, s_ref, p_ref):
    s = s_ref[0]
    si = lax.bitcast_convert_type(s, jnp.int32)
    u = lax.bitcast_convert_type(si ^ ((si >> 31) | _I32_MIN), jnp.uint32)

    def step(i, cand):
        test = cand | (jnp.uint32(1) << (jnp.uint32(31) - i.astype(jnp.uint32)))
        cnt = jnp.sum((u >= test).astype(jnp.int32))
        return lax.select(cnt >= k_keep, test, cand)

    thr = lax.fori_loop(0, 32, step, jnp.uint32(0))
    cnt_gt = jnp.sum((u > thr).astype(jnp.int32))
    p_ref[0, 0, 0] = thr
    p_ref[0, 0, 1] = (k_keep - cnt_gt).astype(jnp.uint32)


def _sc_body(seq, k_keep, rows_per_core, x_hbm, s_hbm, p_hbm, tok_hbm,
             idx_hbm, s_v, p_v, idxbuf_v, gidx_v, gidx2_v, rows_v, gsem):
    c = lax.axis_index("c")
    s = lax.axis_index("s")
    n_sub = 16
    tiles_per_row = n_sub // rows_per_core
    rows_per_tile = k_keep // tiles_per_row
    n_chunk = seq // LANES

    @pl.when(s < rows_per_core)
    def _select():
        b = c * rows_per_core + s
        pltpu.sync_copy(s_hbm.at[b], s_v)
        pltpu.sync_copy(p_hbm.at[b], p_v)
        pv = p_v[pl.ds(0, LANES)]
        thr = pv[0]
        need = pv[1].astype(jnp.int32)
        zero = jnp.zeros((LANES,), jnp.int32)
        one = jnp.ones((LANES,), jnp.int32)

        def chunk(i, carry):
            off, tie = carry
            sv = s_v[pl.ds(i * LANES, LANES)]
            si = plsc.bitcast(sv, jnp.int32)
            uv = plsc.bitcast(si ^ ((si >> 31) | _I32_MIN), jnp.uint32)
            gt = uv > thr
            eq = uv == thr
            eqi = jnp.where(eq, one, zero)
            eqc = plsc.cumsum(eqi)
            keep = gt | (eq & ((tie + eqc) <= need))
            keepi = jnp.where(keep, one, zero)
            kc = plsc.cumsum(keepi)
            pos = off + kc - 1
            gidx = lax.iota(jnp.int32, LANES) + i * LANES
            plsc.store_scatter(idxbuf_v, [pos], gidx, mask=keep)
            return (off + jnp.sum(keepi), tie + jnp.sum(eqi))

        lax.fori_loop(0, n_chunk, chunk, (jnp.int32(0), jnp.int32(0)))
        pltpu.sync_copy(idxbuf_v, idx_hbm.at[b])

    plsc.subcore_barrier()

    b = c * rows_per_core + s // tiles_per_row
    base = (s % tiles_per_row) * rows_per_tile
    pltpu.sync_copy(idx_hbm.at[b, pl.ds(base, rows_per_tile)], gidx_v)

    boff = b * seq

    def addb(j, _):
        gidx2_v[pl.ds(j * LANES, LANES)] = (
            gidx_v[pl.ds(j * LANES, LANES)] + boff)
        return 0

    lax.fori_loop(0, rows_per_tile // LANES, addb, 0)

    for t in range(rows_per_tile // GCH):
        src = x_hbm.at[gidx2_v.at[pl.ds(t * GCH, GCH)]]
        pltpu.async_copy(src, rows_v, gsem).wait()
        pltpu.sync_copy(rows_v, tok_hbm.at[b, pl.ds(base + t * GCH, GCH)])


def kernel(x, W1, b1, W2, b2):
    batch, seq, dim = x.shape
    if seq <= MIN_SEQ_LEN:
        return x, None
    k_keep = max(1, int(seq * TARGET_RATIO))

    h = jax.nn.gelu(jnp.einsum("bsd,dh->bsh", x, W1) + b1, approximate=False)
    scores = (jnp.einsum("bsh,ho->bso", h, W2) + b2)[..., 0]

    params = pl.pallas_call(
        functools.partial(_params_body, k_keep),
        grid=(batch,),
        in_specs=[pl.BlockSpec((1, 1, seq), lambda b: (b, 0, 0))],
        out_specs=pl.BlockSpec((1, 1, 16), lambda b: (b, 0, 0),
                               memory_space=pltpu.SMEM),
        out_shape=jax.ShapeDtypeStruct((batch, 1, 16), jnp.uint32),
    )(scores.reshape(batch, 1, seq))
    params = params.reshape(batch, 16)

    info = plsc.get_sparse_core_info()
    rows_per_core = batch // info.num_cores
    rows_per_tile = k_keep // (16 // rows_per_core)
    mesh = plsc.VectorSubcoreMesh(core_axis_name="c", subcore_axis_name="s")

    sc = pl.kernel(
        functools.partial(_sc_body, seq, k_keep, rows_per_core),
        out_type=[
            jax.ShapeDtypeStruct((batch, k_keep, dim), jnp.float32),
            jax.ShapeDtypeStruct((batch, k_keep), jnp.int32),
        ],
        mesh=mesh,
        compiler_params=pltpu.CompilerParams(needs_layout_passes=False),
        scratch_types=[
            pltpu.VMEM((seq,), jnp.float32),
            pltpu.VMEM((16,), jnp.uint32),
            pltpu.VMEM((k_keep,), jnp.int32),
            pltpu.VMEM((rows_per_tile,), jnp.int32),
            pltpu.VMEM((rows_per_tile,), jnp.int32),
            pltpu.VMEM((GCH, dim), jnp.float32),
            pltpu.SemaphoreType.DMA,
        ],
    )
    tokens, idx = sc(x.reshape(batch * seq, dim), scores, params)
    return tokens, idx


# speedup vs baseline: 1.3040x; 1.1102x over previous
import functools

import jax
import jax.numpy as jnp
from jax import lax
from jax.experimental import pallas as pl
from jax.experimental.pallas import tpu as pltpu
from jax.experimental.pallas import tpu_sc as plsc

HIDDEN = 128
TARGET_RATIO = 0.5
MIN_SEQ_LEN = 2048
LANES = 16
GCH = 64

_I32_MIN = -2147483648


def _params_body(k_keep, batch, s_ref, p_ref):
    s = s_ref[...]
    si = lax.bitcast_convert_type(s, jnp.int32)
    u = lax.bitcast_convert_type(si ^ ((si >> 31) | _I32_MIN), jnp.uint32)

    def step(i, cand):
        test = cand | (jnp.uint32(1) << (jnp.uint32(31) - i.astype(jnp.uint32)))
        cnt = jnp.sum((u >= test).astype(jnp.int32), axis=1, keepdims=True)
        return jnp.where(cnt >= k_keep, test, cand)

    thr = lax.fori_loop(0, 32, step, jnp.zeros((batch, 1), jnp.uint32))
    cnt_gt = jnp.sum((u > thr).astype(jnp.int32), axis=1, keepdims=True)
    need = (k_keep - cnt_gt).astype(jnp.uint32)
    lane = lax.broadcasted_iota(jnp.uint32, (batch, 128), 1)
    p_ref[...] = (jnp.where(lane == 0, thr, jnp.uint32(0))
                  | jnp.where(lane == 1, need, jnp.uint32(0)))


def _sc_body(seq, k_keep, rows_per_core, x_hbm, s_hbm, p_hbm, tok_hbm,
             idx_hbm, s_v, p_v, idxbuf_v, gidx_v, gidx2_v, rows_v, gsem):
    c = lax.axis_index("c")
    s = lax.axis_index("s")
    n_sub = 16
    tiles_per_row = n_sub // rows_per_core
    rows_per_tile = k_keep // tiles_per_row
    n_chunk = seq // LANES

    @pl.when(s < rows_per_core)
    def _select():
        b = c * rows_per_core + s
        pltpu.sync_copy(s_hbm.at[b], s_v)
        pltpu.sync_copy(p_hbm.at[b], p_v)
        pv = p_v[pl.ds(0, LANES)]
        thr = pv[0]
        need = pv[1].astype(jnp.int32)
        zero = jnp.zeros((LANES,), jnp.int32)
        one = jnp.ones((LANES,), jnp.int32)

        def chunk(i, carry):
            off, tie = carry
            sv = s_v[pl.ds(i * LANES, LANES)]
            si = plsc.bitcast(sv, jnp.int32)
            uv = plsc.bitcast(si ^ ((si >> 31) | _I32_MIN), jnp.uint32)
            gt = uv > thr
            eq = uv == thr
            eqi = jnp.where(eq, one, zero)
            eqc = plsc.cumsum(eqi)
            keep = gt | (eq & ((tie + eqc) <= need))
            keepi = jnp.where(keep, one, zero)
            kc = plsc.cumsum(keepi)
            pos = off + kc - 1
            gidx = lax.iota(jnp.int32, LANES) + i * LANES
            plsc.store_scatter(idxbuf_v, [pos], gidx, mask=keep)
            return (off + jnp.sum(keepi), tie + jnp.sum(eqi))

        lax.fori_loop(0, n_chunk, chunk, (jnp.int32(0), jnp.int32(0)))
        pltpu.sync_copy(idxbuf_v, idx_hbm.at[b])

    plsc.subcore_barrier()

    b = c * rows_per_core + s // tiles_per_row
    base = (s % tiles_per_row) * rows_per_tile
    pltpu.sync_copy(idx_hbm.at[b, pl.ds(base, rows_per_tile)], gidx_v)

    boff = b * seq

    def addb(j, _):
        gidx2_v[pl.ds(j * LANES, LANES)] = (
            gidx_v[pl.ds(j * LANES, LANES)] + boff)
        return 0

    lax.fori_loop(0, rows_per_tile // LANES, addb, 0)

    for t in range(rows_per_tile // GCH):
        src = x_hbm.at[gidx2_v.at[pl.ds(t * GCH, GCH)]]
        pltpu.async_copy(src, rows_v, gsem).wait()
        pltpu.sync_copy(rows_v, tok_hbm.at[b, pl.ds(base + t * GCH, GCH)])


def kernel(x, W1, b1, W2, b2):
    batch, seq, dim = x.shape
    if seq <= MIN_SEQ_LEN:
        return x, None
    k_keep = max(1, int(seq * TARGET_RATIO))

    h = jax.nn.gelu(jnp.einsum("bsd,dh->bsh", x, W1) + b1, approximate=False)
    scores = (jnp.einsum("bsh,ho->bso", h, W2) + b2)[..., 0]

    params = pl.pallas_call(
        functools.partial(_params_body, k_keep, batch),
        out_shape=jax.ShapeDtypeStruct((batch, 128), jnp.uint32),
    )(scores)

    info = plsc.get_sparse_core_info()
    rows_per_core = batch // info.num_cores
    rows_per_tile = k_keep // (16 // rows_per_core)
    mesh = plsc.VectorSubcoreMesh(core_axis_name="c", subcore_axis_name="s")

    sc = pl.kernel(
        functools.partial(_sc_body, seq, k_keep, rows_per_core),
        out_type=[
            jax.ShapeDtypeStruct((batch, k_keep, dim), jnp.float32),
            jax.ShapeDtypeStruct((batch, k_keep), jnp.int32),
        ],
        mesh=mesh,
        compiler_params=pltpu.CompilerParams(needs_layout_passes=False),
        scratch_types=[
            pltpu.VMEM((seq,), jnp.float32),
            pltpu.VMEM((128,), jnp.uint32),
            pltpu.VMEM((k_keep,), jnp.int32),
            pltpu.VMEM((rows_per_tile,), jnp.int32),
            pltpu.VMEM((rows_per_tile,), jnp.int32),
            pltpu.VMEM((GCH, dim), jnp.float32),
            pltpu.SemaphoreType.DMA,
        ],
    )
    tokens, idx = sc(x.reshape(batch * seq, dim), scores, params)
    return tokens, idx


# speedup vs baseline: 1.3270x; 1.0177x over previous
import functools

import jax
import jax.numpy as jnp
from jax import lax
from jax.experimental import pallas as pl
from jax.experimental.pallas import tpu as pltpu
from jax.experimental.pallas import tpu_sc as plsc

HIDDEN = 128
TARGET_RATIO = 0.5
MIN_SEQ_LEN = 2048
LANES = 16
GCH = 64

_I32_MIN = -2147483648


def _params_body(k_keep, batch, s_ref, p_ref):
    s = s_ref[...]
    si = lax.bitcast_convert_type(s, jnp.int32)
    u = lax.bitcast_convert_type(si ^ ((si >> 31) | _I32_MIN), jnp.uint32)

    def step(i, cand):
        test = cand | (jnp.uint32(1) << (jnp.uint32(31) - i.astype(jnp.uint32)))
        cnt = jnp.sum((u >= test).astype(jnp.int32), axis=1, keepdims=True)
        return jnp.where(cnt >= k_keep, test, cand)

    thr = lax.fori_loop(0, 32, step, jnp.zeros((batch, 1), jnp.uint32))
    cnt_gt = jnp.sum((u > thr).astype(jnp.int32), axis=1, keepdims=True)
    need = (k_keep - cnt_gt).astype(jnp.uint32)
    lane = lax.broadcasted_iota(jnp.uint32, (batch, 128), 1)
    p_ref[...] = (jnp.where(lane == 0, thr, jnp.uint32(0))
                  | jnp.where(lane == 1, need, jnp.uint32(0)))


def _sc_body(seq, k_keep, rows_per_core, x_hbm, s_hbm, p_hbm, tok_hbm,
             idx_hbm, s_v, p_v, idxbuf_v, gidx_v, gidx2_v, rows_a, rows_b,
             gsem_a, gsem_b, osem_a, osem_b):
    c = lax.axis_index("c")
    s = lax.axis_index("s")
    n_sub = 16
    tiles_per_row = n_sub // rows_per_core
    rows_per_tile = k_keep // tiles_per_row
    n_chunk = seq // LANES

    @pl.when(s < rows_per_core)
    def _select():
        b = c * rows_per_core + s
        pltpu.sync_copy(s_hbm.at[b], s_v)
        pltpu.sync_copy(p_hbm.at[b], p_v)
        pv = p_v[pl.ds(0, LANES)]
        thr = pv[0]
        need = pv[1].astype(jnp.int32)
        zero = jnp.zeros((LANES,), jnp.int32)
        one = jnp.ones((LANES,), jnp.int32)

        def chunk(i, carry):
            off, tie = carry
            sv = s_v[pl.ds(i * LANES, LANES)]
            si = plsc.bitcast(sv, jnp.int32)
            uv = plsc.bitcast(si ^ ((si >> 31) | _I32_MIN), jnp.uint32)
            gt = uv > thr
            eq = uv == thr
            eqi = jnp.where(eq, one, zero)
            eqc = plsc.cumsum(eqi)
            keep = gt | (eq & ((tie + eqc) <= need))
            keepi = jnp.where(keep, one, zero)
            kc = plsc.cumsum(keepi)
            pos = off + kc - 1
            gidx = lax.iota(jnp.int32, LANES) + i * LANES
            plsc.store_scatter(idxbuf_v, [pos], gidx, mask=keep)
            return (off + jnp.sum(keepi), tie + jnp.sum(eqi))

        lax.fori_loop(0, n_chunk, chunk, (jnp.int32(0), jnp.int32(0)))
        pltpu.sync_copy(idxbuf_v, idx_hbm.at[b])

    plsc.subcore_barrier()

    b = c * rows_per_core + s // tiles_per_row
    base = (s % tiles_per_row) * rows_per_tile
    pltpu.sync_copy(idx_hbm.at[b, pl.ds(base, rows_per_tile)], gidx_v)

    boff = b * seq

    def addb(j, _):
        gidx2_v[pl.ds(j * LANES, LANES)] = (
            gidx_v[pl.ds(j * LANES, LANES)] + boff)
        return 0

    lax.fori_loop(0, rows_per_tile // LANES, addb, 0)

    nch = rows_per_tile // GCH
    bufs = (rows_a, rows_b)
    gsems = (gsem_a, gsem_b)
    osems = (osem_a, osem_b)

    def g_src(t):
        return x_hbm.at[gidx2_v.at[pl.ds(t * GCH, GCH)]]

    def o_dst(t):
        return tok_hbm.at[b, pl.ds(base + t * GCH, GCH)]

    gcp = {0: pltpu.async_copy(g_src(0), bufs[0], gsems[0])}
    ocp = {}
    for t in range(nch):
        pb = t & 1
        gcp[t].wait()
        if t >= 1:
            ocp[t - 1].wait()
        if t + 1 < nch:
            gcp[t + 1] = pltpu.async_copy(
                g_src(t + 1), bufs[(t + 1) & 1], gsems[(t + 1) & 1])
        ocp[t] = pltpu.async_copy(bufs[pb], o_dst(t), osems[pb])
    ocp[nch - 1].wait()


def kernel(x, W1, b1, W2, b2):
    batch, seq, dim = x.shape
    if seq <= MIN_SEQ_LEN:
        return x, None
    k_keep = max(1, int(seq * TARGET_RATIO))

    h = jax.nn.gelu(jnp.einsum("bsd,dh->bsh", x, W1) + b1, approximate=False)
    scores = (jnp.einsum("bsh,ho->bso", h, W2) + b2)[..., 0]

    params = pl.pallas_call(
        functools.partial(_params_body, k_keep, batch),
        out_shape=jax.ShapeDtypeStruct((batch, 128), jnp.uint32),
    )(scores)

    info = plsc.get_sparse_core_info()
    rows_per_core = batch // info.num_cores
    rows_per_tile = k_keep // (16 // rows_per_core)
    mesh = plsc.VectorSubcoreMesh(core_axis_name="c", subcore_axis_name="s")

    sc = pl.kernel(
        functools.partial(_sc_body, seq, k_keep, rows_per_core),
        out_type=[
            jax.ShapeDtypeStruct((batch, k_keep, dim), jnp.float32),
            jax.ShapeDtypeStruct((batch, k_keep), jnp.int32),
        ],
        mesh=mesh,
        compiler_params=pltpu.CompilerParams(needs_layout_passes=False),
        scratch_types=[
            pltpu.VMEM((seq,), jnp.float32),
            pltpu.VMEM((128,), jnp.uint32),
            pltpu.VMEM((k_keep,), jnp.int32),
            pltpu.VMEM((rows_per_tile,), jnp.int32),
            pltpu.VMEM((rows_per_tile,), jnp.int32),
            pltpu.VMEM((GCH, dim), jnp.float32),
            pltpu.VMEM((GCH, dim), jnp.float32),
            pltpu.SemaphoreType.DMA,
            pltpu.SemaphoreType.DMA,
            pltpu.SemaphoreType.DMA,
            pltpu.SemaphoreType.DMA,
        ],
    )
    tokens, idx = sc(x.reshape(batch * seq, dim), scores, params)
    return tokens, idx


# speedup vs baseline: 1.3553x; 1.0213x over previous
import functools

import jax
import jax.numpy as jnp
from jax import lax
from jax.experimental import pallas as pl
from jax.experimental.pallas import tpu as pltpu
from jax.experimental.pallas import tpu_sc as plsc

HIDDEN = 128
TARGET_RATIO = 0.5
MIN_SEQ_LEN = 2048
LANES = 16
GCH = 64

_I32_MIN = -2147483648


def _params_body(k_keep, batch, s_ref, p_ref):
    s = s_ref[...]
    si = lax.bitcast_convert_type(s, jnp.int32)
    u = lax.bitcast_convert_type(si ^ ((si >> 31) | _I32_MIN), jnp.uint32)

    def step(i, cand):
        test = cand | (jnp.uint32(1) << (jnp.uint32(31) - i.astype(jnp.uint32)))
        cnt = jnp.sum((u >= test).astype(jnp.int32), axis=1, keepdims=True)
        return jnp.where(cnt >= k_keep, test, cand)

    thr = lax.fori_loop(0, 32, step, jnp.zeros((batch, 1), jnp.uint32))
    cnt_gt = jnp.sum((u > thr).astype(jnp.int32), axis=1, keepdims=True)
    need = (k_keep - cnt_gt).astype(jnp.uint32)
    lane = lax.broadcasted_iota(jnp.uint32, (batch, 128), 1)
    p_ref[...] = (jnp.where(lane == 0, thr, jnp.uint32(0))
                  | jnp.where(lane == 1, need, jnp.uint32(0)))


def _sc_body(seq, k_keep, rows_per_core, x_hbm, s_hbm, p_hbm, tok_hbm,
             idx_hbm, s_v, p_v, idxbuf_v, idxbuf2_v, gidx_v, gidx2_v, rows_a,
             rows_b, gsem_a, gsem_b, osem_a, osem_b):
    c = lax.axis_index("c")
    s = lax.axis_index("s")
    n_sub = 16
    tiles_per_row = n_sub // rows_per_core
    rows_per_tile = k_keep // tiles_per_row
    n_chunk = seq // LANES

    @pl.when(s < rows_per_core)
    def _select():
        b = c * rows_per_core + s
        pltpu.sync_copy(s_hbm.at[b], s_v)
        pltpu.sync_copy(p_hbm.at[b], p_v)
        pv = p_v[pl.ds(0, LANES)]
        thr = pv[0]
        need = pv[1].astype(jnp.int32)
        zero = jnp.zeros((LANES,), jnp.int32)
        one = jnp.ones((LANES,), jnp.int32)

        def chunk(i, off):
            sv = s_v[pl.ds(i * LANES, LANES)]
            si = plsc.bitcast(sv, jnp.int32)
            uv = plsc.bitcast(si ^ ((si >> 31) | _I32_MIN), jnp.uint32)
            ge = uv >= thr
            gc = plsc.cumsum(jnp.where(ge, one, zero))
            pos = off + gc - 1
            gidx = lax.iota(jnp.int32, LANES) + i * LANES
            plsc.store_scatter(idxbuf_v, [pos], gidx, mask=ge)
            return off + gc[15]

        cnt_ge = lax.fori_loop(0, n_chunk, chunk, jnp.int32(0))

        @pl.when(cnt_ge > k_keep)
        def _fixup():
            def fchunk(j, carry):
                off2, tie = carry
                valid = (lax.iota(jnp.int32, LANES) + j * LANES) < cnt_ge
                cand = idxbuf_v[pl.ds(j * LANES, LANES)]
                svv = plsc.load_gather(s_v, [cand], mask=valid)
                siv = plsc.bitcast(svv, jnp.int32)
                uvv = plsc.bitcast(siv ^ ((siv >> 31) | _I32_MIN), jnp.uint32)
                eq = (uvv == thr) & valid
                eqc = plsc.cumsum(jnp.where(eq, one, zero))
                keep2 = valid & ~(eq & ((tie + eqc) > need))
                kc2 = plsc.cumsum(jnp.where(keep2, one, zero))
                pos2 = off2 + kc2 - 1
                plsc.store_scatter(idxbuf2_v, [pos2], cand, mask=keep2)
                return (off2 + kc2[15], tie + eqc[15])

            n2 = (cnt_ge + LANES - 1) // LANES
            lax.fori_loop(0, n2, fchunk, (jnp.int32(0), jnp.int32(0)))
            pltpu.sync_copy(idxbuf2_v.at[pl.ds(0, k_keep)], idx_hbm.at[b])

        @pl.when(cnt_ge <= k_keep)
        def _direct():
            pltpu.sync_copy(idxbuf_v.at[pl.ds(0, k_keep)], idx_hbm.at[b])

    plsc.subcore_barrier()

    b = c * rows_per_core + s // tiles_per_row
    base = (s % tiles_per_row) * rows_per_tile
    pltpu.sync_copy(idx_hbm.at[b, pl.ds(base, rows_per_tile)], gidx_v)

    boff = b * seq

    def addb(j, _):
        gidx2_v[pl.ds(j * LANES, LANES)] = (
            gidx_v[pl.ds(j * LANES, LANES)] + boff)
        return 0

    lax.fori_loop(0, rows_per_tile // LANES, addb, 0)

    nch = rows_per_tile // GCH
    bufs = (rows_a, rows_b)
    gsems = (gsem_a, gsem_b)
    osems = (osem_a, osem_b)

    def g_src(t):
        return x_hbm.at[gidx2_v.at[pl.ds(t * GCH, GCH)]]

    def o_dst(t):
        return tok_hbm.at[b, pl.ds(base + t * GCH, GCH)]

    gcp = {0: pltpu.async_copy(g_src(0), bufs[0], gsems[0])}
    ocp = {}
    for t in range(nch):
        pb = t & 1
        gcp[t].wait()
        if t >= 1:
            ocp[t - 1].wait()
        if t + 1 < nch:
            gcp[t + 1] = pltpu.async_copy(
                g_src(t + 1), bufs[(t + 1) & 1], gsems[(t + 1) & 1])
        ocp[t] = pltpu.async_copy(bufs[pb], o_dst(t), osems[pb])
    ocp[nch - 1].wait()


def kernel(x, W1, b1, W2, b2):
    batch, seq, dim = x.shape
    if seq <= MIN_SEQ_LEN:
        return x, None
    k_keep = max(1, int(seq * TARGET_RATIO))

    h = jax.nn.gelu(jnp.einsum("bsd,dh->bsh", x, W1) + b1, approximate=False)
    scores = (jnp.einsum("bsh,ho->bso", h, W2) + b2)[..., 0]

    params = pl.pallas_call(
        functools.partial(_params_body, k_keep, batch),
        out_shape=jax.ShapeDtypeStruct((batch, 128), jnp.uint32),
    )(scores)

    info = plsc.get_sparse_core_info()
    rows_per_core = batch // info.num_cores
    rows_per_tile = k_keep // (16 // rows_per_core)
    mesh = plsc.VectorSubcoreMesh(core_axis_name="c", subcore_axis_name="s")

    sc = pl.kernel(
        functools.partial(_sc_body, seq, k_keep, rows_per_core),
        out_type=[
            jax.ShapeDtypeStruct((batch, k_keep, dim), jnp.float32),
            jax.ShapeDtypeStruct((batch, k_keep), jnp.int32),
        ],
        mesh=mesh,
        compiler_params=pltpu.CompilerParams(needs_layout_passes=False),
        scratch_types=[
            pltpu.VMEM((seq,), jnp.float32),
            pltpu.VMEM((128,), jnp.uint32),
            pltpu.VMEM((seq,), jnp.int32),
            pltpu.VMEM((k_keep,), jnp.int32),
            pltpu.VMEM((rows_per_tile,), jnp.int32),
            pltpu.VMEM((rows_per_tile,), jnp.int32),
            pltpu.VMEM((GCH, dim), jnp.float32),
            pltpu.VMEM((GCH, dim), jnp.float32),
            pltpu.SemaphoreType.DMA,
            pltpu.SemaphoreType.DMA,
            pltpu.SemaphoreType.DMA,
            pltpu.SemaphoreType.DMA,
        ],
    )
    tokens, idx = sc(x.reshape(batch * seq, dim), scores, params)
    return tokens, idx
